# Initial kernel scaffold; baseline (speedup 1.0000x reference)
import jax
import jax.numpy as jnp
from jax.experimental import pallas as pl


def kernel(x, edge_index, edge_attr, W, att_src, att_dst, W_edge, att_edge, bias):
    raise NotImplementedError("write your pallas kernel here")


# speedup vs baseline: 25.0960x; 25.0960x over previous
import functools

import jax
import jax.numpy as jnp
from jax import lax
from jax.experimental import pallas as pl
from jax.experimental.pallas import tpu as pltpu
from jax.experimental.pallas import tpu_sc as plsc

N = 10000
E = 320000
D = 128

NC = 2
NS = 16
LANES = 16
NW = NC * NS
EPW = E // NW
K = 80
NCHUNK = EPW // K
NPAD = 10240
RPS = NPAD // NS
BLK = 400
NBLK = N // BLK
BLK2 = 1280
NBLK2 = NPAD // BLK2

_mesh = plsc.VectorSubcoreMesh(
    core_axis_name="c", subcore_axis_name="s", num_cores=NC, num_subcores=NS
)


def _proj_body(x_ref, w_ref, asrc_ref, adst_ref, h_ref, as_ref, ad_ref):
    h = jnp.dot(x_ref[...], w_ref[...], preferred_element_type=jnp.float32)
    h_ref[...] = h
    as_ref[...] = jnp.sum(h * asrc_ref[...], axis=1).reshape(1, 1, BLK)
    ad_ref[...] = jnp.sum(h * adst_ref[...], axis=1).reshape(1, 1, BLK)


_proj = pl.pallas_call(
    _proj_body,
    grid=(NBLK,),
    in_specs=[
        pl.BlockSpec((BLK, D), lambda i: (i, 0)),
        pl.BlockSpec((D, D), lambda i: (0, 0)),
        pl.BlockSpec((1, D), lambda i: (0, 0)),
        pl.BlockSpec((1, D), lambda i: (0, 0)),
    ],
    out_specs=[
        pl.BlockSpec((BLK, D), lambda i: (i, 0)),
        pl.BlockSpec((1, 1, BLK), lambda i: (i, 0, 0)),
        pl.BlockSpec((1, 1, BLK), lambda i: (i, 0, 0)),
    ],
    out_shape=[
        jax.ShapeDtypeStruct((N, D), jnp.float32),
        jax.ShapeDtypeStruct((NBLK, 1, BLK), jnp.float32),
        jax.ShapeDtypeStruct((NBLK, 1, BLK), jnp.float32),
    ],
)


def _sc_body(as_h, ad_h, em_h, cv_h, sh_h, h_h,
             accp_h, denp_h,
             asb, adb, denb, emb, srcb, dstb, rows, cvb, shb,
             acc, sem):
    cid = lax.axis_index("c")
    sid = lax.axis_index("s")
    wid = sid * NC + cid

    pltpu.sync_copy(as_h, asb)
    pltpu.sync_copy(ad_h, adb)
    pltpu.sync_copy(cv_h, cvb)
    pltpu.sync_copy(sh_h, shb)

    z16 = jnp.zeros((LANES,), jnp.float32)

    def zden(i, carry):
        denb[pl.ds(i * LANES, LANES)] = z16
        return carry

    lax.fori_loop(0, NPAD // LANES, zden, 0)

    for j in range(K):
        for v in range(D // LANES):
            rows[j, pl.ds(v * LANES, LANES)] = z16
    for t in range(RPS // K):
        pltpu.sync_copy(rows, acc.at[pl.ds(sid * RPS + t * K, K)])
    plsc.subcore_barrier()

    cv = cvb[...]
    sh = shb[...]

    def chunk(i, carry):
        pltpu.sync_copy(em_h.at[wid, i], emb)
        for g in range(K // LANES):
            sl = pl.ds(g * LANES, LANES)
            srcb[sl] = emb[0, sl]
            dstb[sl] = emb[1, sl]
        pltpu.async_copy(h_h.at[srcb], rows, sem).wait()
        for g in range(K // LANES):
            sl = pl.ds(g * LANES, LANES)
            si = emb[0, sl]
            di = emb[1, sl]
            a = plsc.load_gather(asb, [si]) + plsc.load_gather(adb, [di])
            a = a + plsc.bitcast(emb[2, sl], jnp.float32) * cv
            a = jnp.where(a >= 0.0, a, 0.2 * a)
            ex = jnp.exp(a - sh)
            plsc.addupdate_scatter(denb, [di], ex)
            for lane in range(LANES):
                s = ex[lane]
                j = g * LANES + lane
                for v in range(D // LANES):
                    sv = pl.ds(v * LANES, LANES)
                    rows[j, sv] = rows[j, sv] * s
        pltpu.sync_copy(rows, acc.at[dstb], add=True)
        return carry

    lax.fori_loop(0, NCHUNK, chunk, 0)

    pltpu.sync_copy(denb, denp_h.at[wid])
    plsc.subcore_barrier()
    for t in range(RPS // K):
        r0 = sid * RPS + t * K
        pltpu.sync_copy(acc.at[pl.ds(r0, K)], accp_h.at[cid, pl.ds(r0, K)])


_sc_kernel = pl.kernel(
    _sc_body,
    out_type=(
        jax.ShapeDtypeStruct((NC, NPAD, D), jnp.float32),
        jax.ShapeDtypeStruct((NW, NPAD), jnp.float32),
    ),
    mesh=_mesh,
    compiler_params=pltpu.CompilerParams(needs_layout_passes=False),
    scratch_types=(
        pltpu.VMEM((N,), jnp.float32),
        pltpu.VMEM((N,), jnp.float32),
        pltpu.VMEM((NPAD,), jnp.float32),
        pltpu.VMEM((3, K), jnp.int32),
        pltpu.VMEM((K,), jnp.int32),
        pltpu.VMEM((K,), jnp.int32),
        pltpu.VMEM((K, D), jnp.float32),
        pltpu.VMEM((LANES,), jnp.float32),
        pltpu.VMEM((LANES,), jnp.float32),
        pltpu.VMEM_SHARED((NPAD, D), jnp.float32),
        pltpu.SemaphoreType.DMA,
    ),
)


def _fin_body(ap_ref, dp_ref, b_ref, o_ref):
    acc = ap_ref[0] + ap_ref[1]
    den = jnp.sum(dp_ref[...], axis=0)
    o_ref[...] = jnp.maximum(acc / (den[:, None] + 1e-16) + b_ref[...], 0.0)


_fin = pl.pallas_call(
    _fin_body,
    grid=(NBLK2,),
    in_specs=[
        pl.BlockSpec((NC, BLK2, D), lambda i: (0, i, 0)),
        pl.BlockSpec((NW, BLK2), lambda i: (0, i)),
        pl.BlockSpec((1, D), lambda i: (0, 0)),
    ],
    out_specs=pl.BlockSpec((BLK2, D), lambda i: (i, 0)),
    out_shape=jax.ShapeDtypeStruct((NPAD, D), jnp.float32),
)


@jax.jit
def kernel(x, edge_index, edge_attr, W, att_src, att_dst, W_edge, att_edge, bias):
    attr = edge_attr.reshape(E)
    abits = lax.bitcast_convert_type(attr, jnp.int32)
    em = jnp.concatenate(
        [
            edge_index[0].reshape(NW, NCHUNK, 1, K),
            edge_index[1].reshape(NW, NCHUNK, 1, K),
            abits.reshape(NW, NCHUNK, 1, K),
        ],
        axis=2,
    )

    h, as3, ad3 = _proj(x, W, att_src.reshape(1, D), att_dst.reshape(1, D))
    a_s = as3.reshape(N)
    a_d = ad3.reshape(N)

    c = jnp.vdot(W_edge[0], att_edge)
    maxsum = jnp.max(a_s) + jnp.max(a_d) + jnp.abs(c) * jnp.max(jnp.abs(attr))
    shift = jnp.where(maxsum > 0, maxsum, 0.2 * maxsum)
    cvec = jnp.full((LANES,), c, jnp.float32)
    shvec = jnp.full((LANES,), shift, jnp.float32)

    accp, denp = _sc_kernel(a_s, a_d, em, cvec, shvec, h)

    out = _fin(accp, denp, bias.reshape(1, D))
    return out[:N]


# speedup vs baseline: 47.2350x; 1.8822x over previous
import functools

import jax
import jax.numpy as jnp
from jax import lax
from jax.experimental import pallas as pl
from jax.experimental.pallas import tpu as pltpu
from jax.experimental.pallas import tpu_sc as plsc

N = 10000
E = 320000
D = 128

NC = 2
NS = 16
LANES = 16
NW = NC * NS
EPW = E // NW
K = 80
NCHUNK = EPW // K
NPAD = 10240
RPS = NPAD // NS
BLK = 400
NBLK = N // BLK
BLK2 = 1280
NBLK2 = NPAD // BLK2

_mesh = plsc.VectorSubcoreMesh(
    core_axis_name="c", subcore_axis_name="s", num_cores=NC, num_subcores=NS
)


def _proj_body(x_ref, w_ref, asrc_ref, adst_ref, h_ref, as_ref, ad_ref):
    h = jnp.dot(x_ref[...], w_ref[...], preferred_element_type=jnp.float32)
    h_ref[...] = h
    as_ref[...] = jnp.sum(h * asrc_ref[...], axis=1).reshape(1, 1, BLK)
    ad_ref[...] = jnp.sum(h * adst_ref[...], axis=1).reshape(1, 1, BLK)


_proj = pl.pallas_call(
    _proj_body,
    grid=(NBLK,),
    in_specs=[
        pl.BlockSpec((BLK, D), lambda i: (i, 0)),
        pl.BlockSpec((D, D), lambda i: (0, 0)),
        pl.BlockSpec((1, D), lambda i: (0, 0)),
        pl.BlockSpec((1, D), lambda i: (0, 0)),
    ],
    out_specs=[
        pl.BlockSpec((BLK, D), lambda i: (i, 0)),
        pl.BlockSpec((1, 1, BLK), lambda i: (i, 0, 0)),
        pl.BlockSpec((1, 1, BLK), lambda i: (i, 0, 0)),
    ],
    out_shape=[
        jax.ShapeDtypeStruct((N, D), jnp.float32),
        jax.ShapeDtypeStruct((NBLK, 1, BLK), jnp.float32),
        jax.ShapeDtypeStruct((NBLK, 1, BLK), jnp.float32),
    ],
)


def _pre_body(as_h, ad_h, src_h, dst_h, attr_h, cv_h, sh_h,
              ex_h, denp_h,
              asb, adb, denb, srcw, dstw, aew, cvb, shb):
    cid = lax.axis_index("c")
    sid = lax.axis_index("s")
    wid = sid * NC + cid

    pltpu.sync_copy(as_h, asb)
    pltpu.sync_copy(ad_h, adb)
    pltpu.sync_copy(cv_h, cvb)
    pltpu.sync_copy(sh_h, shb)
    base = pl.ds(wid * EPW, EPW)
    pltpu.sync_copy(src_h.at[base], srcw)
    pltpu.sync_copy(dst_h.at[base], dstw)
    pltpu.sync_copy(attr_h.at[base], aew)

    z16 = jnp.zeros((LANES,), jnp.float32)

    def zden(i, carry):
        denb[pl.ds(i * LANES, LANES)] = z16
        return carry

    lax.fori_loop(0, NPAD // LANES, zden, 0)

    cv = cvb[...]
    sh = shb[...]

    def grp(g, carry):
        sl = pl.ds(g * LANES, LANES)
        si = srcw[sl]
        di = dstw[sl]
        a = plsc.load_gather(asb, [si]) + plsc.load_gather(adb, [di])
        a = a + aew[sl] * cv
        a = jnp.where(a >= 0.0, a, 0.2 * a)
        ex = jnp.exp(a - sh)
        plsc.addupdate_scatter(denb, [di], ex)
        aew[sl] = ex
        return carry

    lax.fori_loop(0, EPW // LANES, grp, 0)

    pltpu.sync_copy(aew, ex_h.at[base])
    pltpu.sync_copy(denb, denp_h.at[wid])


_pre_kernel = pl.kernel(
    _pre_body,
    out_type=(
        jax.ShapeDtypeStruct((E,), jnp.float32),
        jax.ShapeDtypeStruct((NW, NPAD), jnp.float32),
    ),
    mesh=_mesh,
    compiler_params=pltpu.CompilerParams(needs_layout_passes=False),
    scratch_types=(
        pltpu.VMEM((N,), jnp.float32),
        pltpu.VMEM((N,), jnp.float32),
        pltpu.VMEM((NPAD,), jnp.float32),
        pltpu.VMEM((EPW,), jnp.int32),
        pltpu.VMEM((EPW,), jnp.int32),
        pltpu.VMEM((EPW,), jnp.float32),
        pltpu.VMEM((LANES,), jnp.float32),
        pltpu.VMEM((LANES,), jnp.float32),
    ),
)


def _main_body(src_h, dst_h, ex_h, h_h, accp_h, *sc):
    srcbs = sc[0:4]
    dstbs = sc[4:8]
    exbs = sc[8:12]
    rowss = sc[12:16]
    acc = sc[16]
    sems_e = sc[17:21]
    sems_g = sc[21:25]
    sems_s = sc[25:29]

    cid = lax.axis_index("c")
    sid = lax.axis_index("s")
    wid = sid * NC + cid

    z16 = jnp.zeros((LANES,), jnp.float32)
    for j in range(K):
        for v in range(D // LANES):
            rowss[0][j, pl.ds(v * LANES, LANES)] = z16
    for t in range(RPS // K):
        pltpu.sync_copy(rowss[0], acc.at[pl.ds(sid * RPS + t * K, K)])
    plsc.subcore_barrier()

    def em_issue(i, s):
        sl = pl.ds(wid * EPW + i * K, K)
        pltpu.async_copy(src_h.at[sl], srcbs[s], sems_e[s])
        pltpu.async_copy(dst_h.at[sl], dstbs[s], sems_e[s])
        pltpu.async_copy(ex_h.at[sl], exbs[s], sems_e[s])

    def em_wait(i, s):
        sl = pl.ds(wid * EPW + i * K, K)
        pltpu.make_async_copy(src_h.at[sl], srcbs[s], sems_e[s]).wait()
        pltpu.make_async_copy(dst_h.at[sl], dstbs[s], sems_e[s]).wait()
        pltpu.make_async_copy(ex_h.at[sl], exbs[s], sems_e[s]).wait()

    def gather_issue(i, s):
        em_wait(i, s)
        pltpu.async_copy(h_h.at[srcbs[s]], rowss[s], sems_g[s])

    def scatter_wait(s):
        pltpu.make_async_copy(rowss[s], acc.at[dstbs[s]], sems_s[s]).wait()

    def compute(i, s):
        pltpu.make_async_copy(h_h.at[srcbs[s]], rowss[s], sems_g[s]).wait()

        def scale_group(g, carry):
            base = g * LANES
            exv = exbs[s][pl.ds(base, LANES)]
            for lane in range(LANES):
                sj = exv[lane]
                for v in range(D // LANES):
                    sv = pl.ds(v * LANES, LANES)
                    rowss[s][base + lane, sv] = rowss[s][base + lane, sv] * sj
            return carry

        lax.fori_loop(0, K // LANES, scale_group, 0)
        pltpu.async_copy(rowss[s], acc.at[dstbs[s]], sems_s[s], add=True)

    em_issue(0, 0)
    em_issue(1, 1)
    em_issue(2, 2)
    gather_issue(0, 0)
    gather_issue(1, 1)

    def quad(t, carry):
        for p in range(4):
            i = 4 * t + p
            compute(i, p)
            if p >= 2:
                @pl.when(i + 3 <= NCHUNK - 1)
                def _():
                    em_issue(i + 3, (p + 3) % 4)
            else:
                em_issue(i + 3, (p + 3) % 4)
            if p < 2:
                @pl.when(i >= 2)
                def _():
                    scatter_wait((p + 2) % 4)
            else:
                scatter_wait((p + 2) % 4)
            if p == 3:
                @pl.when(i + 2 <= NCHUNK - 1)
                def _():
                    gather_issue(i + 2, (p + 2) % 4)
            else:
                gather_issue(i + 2, (p + 2) % 4)
        return carry

    lax.fori_loop(0, (NCHUNK - 1) // 4, quad, 0)

    compute(NCHUNK - 1, 0)
    scatter_wait(2)
    scatter_wait(3)
    scatter_wait(0)
    plsc.subcore_barrier()
    for t in range(RPS // K):
        r0 = sid * RPS + t * K
        pltpu.sync_copy(acc.at[pl.ds(r0, K)], accp_h.at[cid, pl.ds(r0, K)])


_main_kernel = pl.kernel(
    _main_body,
    out_type=jax.ShapeDtypeStruct((NC, NPAD, D), jnp.float32),
    mesh=_mesh,
    compiler_params=pltpu.CompilerParams(needs_layout_passes=False),
    scratch_types=(
        *[pltpu.VMEM((K,), jnp.int32) for _ in range(4)],
        *[pltpu.VMEM((K,), jnp.int32) for _ in range(4)],
        *[pltpu.VMEM((K,), jnp.float32) for _ in range(4)],
        *[pltpu.VMEM((K, D), jnp.float32) for _ in range(4)],
        pltpu.VMEM_SHARED((NPAD, D), jnp.float32),
        *[pltpu.SemaphoreType.DMA for _ in range(12)],
    ),
)


def _fin_body(ap_ref, dp_ref, b_ref, o_ref):
    acc = ap_ref[0] + ap_ref[1]
    den = jnp.sum(dp_ref[...], axis=0)
    o_ref[...] = jnp.maximum(acc / (den[:, None] + 1e-16) + b_ref[...], 0.0)


_fin = pl.pallas_call(
    _fin_body,
    grid=(NBLK2,),
    in_specs=[
        pl.BlockSpec((NC, BLK2, D), lambda i: (0, i, 0)),
        pl.BlockSpec((NW, BLK2), lambda i: (0, i)),
        pl.BlockSpec((1, D), lambda i: (0, 0)),
    ],
    out_specs=pl.BlockSpec((BLK2, D), lambda i: (i, 0)),
    out_shape=jax.ShapeDtypeStruct((NPAD, D), jnp.float32),
)


@jax.jit
def kernel(x, edge_index, edge_attr, W, att_src, att_dst, W_edge, att_edge, bias):
    attr = edge_attr.reshape(E)
    src1 = edge_index[0]
    dst1 = edge_index[1]

    h, as3, ad3 = _proj(x, W, att_src.reshape(1, D), att_dst.reshape(1, D))
    a_s = as3.reshape(N)
    a_d = ad3.reshape(N)

    c = jnp.vdot(W_edge[0], att_edge)
    maxsum = jnp.max(a_s) + jnp.max(a_d) + jnp.abs(c) * jnp.max(jnp.abs(attr))
    shift = jnp.where(maxsum > 0, maxsum, 0.2 * maxsum)
    cvec = jnp.full((LANES,), c, jnp.float32)
    shvec = jnp.full((LANES,), shift, jnp.float32)

    ex1, denp = _pre_kernel(a_s, a_d, src1, dst1, attr, cvec, shvec)
    accp = _main_kernel(src1, dst1, ex1, h)

    out = _fin(accp, denp, bias.reshape(1, D))
    return out[:N]


# speedup vs baseline: 49.9418x; 1.0573x over previous
import functools

import jax
import jax.numpy as jnp
from jax import lax
from jax.experimental import pallas as pl
from jax.experimental.pallas import tpu as pltpu
from jax.experimental.pallas import tpu_sc as plsc

N = 10000
E = 320000
D = 128

NC = 2
NS = 16
LANES = 16
NW = NC * NS
EPW = E // NW
K = 80
NCHUNK = EPW // K
NPAD = 10240
RPS = NPAD // NS
BLK = 400
NBLK = N // BLK
BLK2 = 1280
NBLK2 = NPAD // BLK2

_mesh = plsc.VectorSubcoreMesh(
    core_axis_name="c", subcore_axis_name="s", num_cores=NC, num_subcores=NS
)


VBLK = 2000


def _vec_body(x_ref, w_ref, asrc_ref, adst_ref, o_ref):
    ws = jnp.sum(w_ref[...] * asrc_ref[...], axis=1)
    wd = jnp.sum(w_ref[...] * adst_ref[...], axis=1)
    wsd = jnp.concatenate([ws[:, None], wd[:, None]], axis=1)
    o_ref[...] = jnp.dot(x_ref[...], wsd, preferred_element_type=jnp.float32)


_vec = pl.pallas_call(
    _vec_body,
    grid=(N // VBLK,),
    in_specs=[
        pl.BlockSpec((VBLK, D), lambda i: (i, 0)),
        pl.BlockSpec((D, D), lambda i: (0, 0)),
        pl.BlockSpec((1, D), lambda i: (0, 0)),
        pl.BlockSpec((1, D), lambda i: (0, 0)),
    ],
    out_specs=pl.BlockSpec((VBLK, 2), lambda i: (i, 0)),
    out_shape=jax.ShapeDtypeStruct((N, 2), jnp.float32),
)


def _proj_body(x_ref, w_ref, h_ref):
    h_ref[...] = jnp.dot(
        x_ref[...], w_ref[...], preferred_element_type=jnp.float32
    )


_proj = pl.pallas_call(
    _proj_body,
    grid=(NBLK,),
    in_specs=[
        pl.BlockSpec((BLK, D), lambda i: (i, 0)),
        pl.BlockSpec((D, D), lambda i: (0, 0)),
    ],
    out_specs=pl.BlockSpec((BLK, D), lambda i: (i, 0)),
    out_shape=jax.ShapeDtypeStruct((N, D), jnp.float32),
)


def _pre_body(asad_h, src_h, dst_h, attr_h, we_h, ae_h,
              ex_h, denp_h,
              tab, denb, srcw, dstw, aew, web, aeb):
    cid = lax.axis_index("c")
    sid = lax.axis_index("s")
    wid = sid * NC + cid

    pltpu.sync_copy(asad_h, tab)
    pltpu.sync_copy(we_h, web)
    pltpu.sync_copy(ae_h, aeb)
    base = pl.ds(wid * EPW, EPW)
    pltpu.sync_copy(src_h.at[base], srcw)
    pltpu.sync_copy(dst_h.at[base], dstw)
    pltpu.sync_copy(attr_h.at[base], aew)

    z16 = jnp.zeros((LANES,), jnp.float32)

    def zden(i, carry):
        denb[pl.ds(i * LANES, LANES)] = z16
        return carry

    lax.fori_loop(0, NPAD // LANES, zden, 0)

    def cred(v, acc):
        sl = pl.ds(v * LANES, LANES)
        return acc + web[sl] * aeb[sl]

    cv_all = lax.fori_loop(0, D // LANES, cred, z16)
    c = jnp.sum(cv_all, axis=0)
    cv = jnp.full((LANES,), c, jnp.float32)

    def mred(i, acc):
        return jnp.maximum(acc, tab[pl.ds(i * LANES, LANES)])

    mv = lax.fori_loop(0, 2 * N // LANES, mred,
                       jnp.full((LANES,), -jnp.inf, jnp.float32))
    maxsum = 2.0 * jnp.max(mv, axis=0)
    shift = jnp.where(maxsum >= 0.0, maxsum, 0.2 * maxsum)
    sh = jnp.full((LANES,), shift, jnp.float32)

    two16 = jnp.full((LANES,), 2, jnp.int32)
    one16 = jnp.full((LANES,), 1, jnp.int32)

    def grp(g, carry):
        sl = pl.ds(g * LANES, LANES)
        si = srcw[sl]
        di = dstw[sl]
        a = plsc.load_gather(tab, [si * two16])
        a = a + plsc.load_gather(tab, [di * two16 + one16])
        a = a + aew[sl] * cv
        a = jnp.where(a >= 0.0, a, 0.2 * a)
        ex = jnp.exp(a - sh)
        plsc.addupdate_scatter(denb, [di], ex)
        aew[sl] = ex
        return carry

    lax.fori_loop(0, EPW // LANES, grp, 0)

    pltpu.sync_copy(aew, ex_h.at[base])
    pltpu.sync_copy(denb, denp_h.at[wid])


_pre_kernel = pl.kernel(
    _pre_body,
    out_type=(
        jax.ShapeDtypeStruct((E,), jnp.float32),
        jax.ShapeDtypeStruct((NW, NPAD), jnp.float32),
    ),
    mesh=_mesh,
    compiler_params=pltpu.CompilerParams(needs_layout_passes=False),
    scratch_types=(
        pltpu.VMEM((2 * N,), jnp.float32),
        pltpu.VMEM((NPAD,), jnp.float32),
        pltpu.VMEM((EPW,), jnp.int32),
        pltpu.VMEM((EPW,), jnp.int32),
        pltpu.VMEM((EPW,), jnp.float32),
        pltpu.VMEM((D,), jnp.float32),
        pltpu.VMEM((D,), jnp.float32),
    ),
)


def _main_body(src_h, dst_h, ex_h, h_h, accp_h, *sc):
    srcbs = sc[0:4]
    dstbs = sc[4:8]
    exbs = sc[8:12]
    rowss = sc[12:16]
    acc = sc[16]
    sems_e = sc[17:21]
    sems_g = sc[21:25]
    sems_s = sc[25:29]

    cid = lax.axis_index("c")
    sid = lax.axis_index("s")
    wid = sid * NC + cid

    z16 = jnp.zeros((LANES,), jnp.float32)
    for j in range(K):
        for v in range(D // LANES):
            rowss[0][j, pl.ds(v * LANES, LANES)] = z16
    for t in range(RPS // K):
        pltpu.sync_copy(rowss[0], acc.at[pl.ds(sid * RPS + t * K, K)])
    plsc.subcore_barrier()

    def em_issue(i, s):
        sl = pl.ds(wid * EPW + i * K, K)
        pltpu.async_copy(src_h.at[sl], srcbs[s], sems_e[s])
        pltpu.async_copy(dst_h.at[sl], dstbs[s], sems_e[s])
        pltpu.async_copy(ex_h.at[sl], exbs[s], sems_e[s])

    def em_wait(i, s):
        sl = pl.ds(wid * EPW + i * K, K)
        pltpu.make_async_copy(src_h.at[sl], srcbs[s], sems_e[s]).wait()
        pltpu.make_async_copy(dst_h.at[sl], dstbs[s], sems_e[s]).wait()
        pltpu.make_async_copy(ex_h.at[sl], exbs[s], sems_e[s]).wait()

    def gather_issue(i, s):
        em_wait(i, s)
        pltpu.async_copy(h_h.at[srcbs[s]], rowss[s], sems_g[s])

    def scatter_wait(s):
        pltpu.make_async_copy(rowss[s], acc.at[dstbs[s]], sems_s[s]).wait()

    def compute(i, s):
        pltpu.make_async_copy(h_h.at[srcbs[s]], rowss[s], sems_g[s]).wait()

        def scale_group(g, carry):
            base = g * LANES
            exv = exbs[s][pl.ds(base, LANES)]
            for lane in range(LANES):
                sj = exv[lane]
                for v in range(D // LANES):
                    sv = pl.ds(v * LANES, LANES)
                    rowss[s][base + lane, sv] = rowss[s][base + lane, sv] * sj
            return carry

        lax.fori_loop(0, K // LANES, scale_group, 0)
        pltpu.async_copy(rowss[s], acc.at[dstbs[s]], sems_s[s], add=True)

    em_issue(0, 0)
    em_issue(1, 1)
    em_issue(2, 2)
    gather_issue(0, 0)
    gather_issue(1, 1)

    def quad(t, carry):
        for p in range(4):
            i = 4 * t + p
            compute(i, p)
            if p >= 2:
                @pl.when(i + 3 <= NCHUNK - 1)
                def _():
                    em_issue(i + 3, (p + 3) % 4)
            else:
                em_issue(i + 3, (p + 3) % 4)
            if p < 2:
                @pl.when(i >= 2)
                def _():
                    scatter_wait((p + 2) % 4)
            else:
                scatter_wait((p + 2) % 4)
            if p == 3:
                @pl.when(i + 2 <= NCHUNK - 1)
                def _():
                    gather_issue(i + 2, (p + 2) % 4)
            else:
                gather_issue(i + 2, (p + 2) % 4)
        return carry

    lax.fori_loop(0, (NCHUNK - 1) // 4, quad, 0)

    compute(NCHUNK - 1, 0)
    scatter_wait(2)
    scatter_wait(3)
    scatter_wait(0)
    plsc.subcore_barrier()
    for t in range(RPS // K):
        r0 = sid * RPS + t * K
        pltpu.sync_copy(acc.at[pl.ds(r0, K)], accp_h.at[cid, pl.ds(r0, K)])


_main_kernel = pl.kernel(
    _main_body,
    out_type=jax.ShapeDtypeStruct((NC, NPAD, D), jnp.float32),
    mesh=_mesh,
    compiler_params=pltpu.CompilerParams(needs_layout_passes=False),
    scratch_types=(
        *[pltpu.VMEM((K,), jnp.int32) for _ in range(4)],
        *[pltpu.VMEM((K,), jnp.int32) for _ in range(4)],
        *[pltpu.VMEM((K,), jnp.float32) for _ in range(4)],
        *[pltpu.VMEM((K, D), jnp.float32) for _ in range(4)],
        pltpu.VMEM_SHARED((NPAD, D), jnp.float32),
        *[pltpu.SemaphoreType.DMA for _ in range(12)],
    ),
)


def _fin_body(ap_ref, dp_ref, b_ref, o_ref):
    acc = ap_ref[0] + ap_ref[1]
    den = jnp.sum(dp_ref[...], axis=0)
    o_ref[...] = jnp.maximum(acc / (den[:, None] + 1e-16) + b_ref[...], 0.0)


_fin = pl.pallas_call(
    _fin_body,
    grid=(NBLK2,),
    in_specs=[
        pl.BlockSpec((NC, BLK2, D), lambda i: (0, i, 0)),
        pl.BlockSpec((NW, BLK2), lambda i: (0, i)),
        pl.BlockSpec((1, D), lambda i: (0, 0)),
    ],
    out_specs=pl.BlockSpec((BLK2, D), lambda i: (i, 0)),
    out_shape=jax.ShapeDtypeStruct((N, D), jnp.float32),
)


@jax.jit
def kernel(x, edge_index, edge_attr, W, att_src, att_dst, W_edge, att_edge, bias):
    attr = edge_attr.reshape(E)
    src1 = edge_index[0]
    dst1 = edge_index[1]

    asad = _vec(x, W, att_src.reshape(1, D), att_dst.reshape(1, D))

    ex1, denp = _pre_kernel(
        asad.reshape(2 * N), src1, dst1, attr,
        W_edge.reshape(D), att_edge,
    )
    h = _proj(x, W)
    accp = _main_kernel(src1, dst1, ex1, h)

    out = _fin(accp, denp, bias.reshape(1, D))
    return out
